# baseline (speedup 1.0000x reference)
import functools

import jax
import jax.numpy as jnp
from jax import lax
from jax.experimental import pallas as pl
from jax.experimental.pallas import tpu as pltpu
from jax.experimental.pallas import tpu_sc as plsc

_L = 16


@functools.lru_cache(maxsize=None)
def _build(B, H, W, D):
    C = 2 * D

    mesh = plsc.VectorSubcoreMesh(
        core_axis_name="c", subcore_axis_name="s", num_cores=1)

    @functools.partial(
        pl.kernel,
        mesh=mesh,
        out_type=jax.ShapeDtypeStruct((B, H, W, C), jnp.float32),
        scratch_types=[
            pltpu.VMEM((W, C), jnp.float32),
            pltpu.VMEM((1, D), jnp.float32),
            pltpu.SemaphoreType.DMA,
        ],
    )
    def pos_embed(row_hbm, col_hbm, out_hbm, plane, rowv, sem):
        w = lax.axis_index("s")

        for half in range(2):
            i = w * 2 + half
            pltpu.sync_copy(
                col_hbm.at[pl.ds(0, W)], plane.at[:, pl.ds(0, D)])
            pltpu.sync_copy(row_hbm.at[pl.ds(i, 1)], rowv)

            segs = [rowv[0, pl.ds(k * _L, _L)] for k in range(D // _L)]
            for j in range(W):
                for k, v in enumerate(segs):
                    plane[j, pl.ds(D + k * _L, _L)] = v

            pltpu.sync_copy(plane, out_hbm.at[0, i])

        plsc.subcore_barrier()

        @pl.when(w < B - 1)
        def _replicate():
            pltpu.sync_copy(
                out_hbm.at[pl.ds(0, 1)], out_hbm.at[pl.ds(w + 1, 1)])

    return pos_embed


def kernel(x, row_embed, col_embed):
    B = x.shape[0]
    H, W = x.shape[-2], x.shape[-1]
    D = row_embed.shape[1]
    out = _build(B, H, W, D)(row_embed, col_embed)
    return jnp.transpose(out, (0, 3, 1, 2))


# speedup vs baseline: 23.0599x; 23.0599x over previous
import functools

import jax
import jax.numpy as jnp
from jax import lax
from jax.experimental import pallas as pl
from jax.experimental.pallas import tpu as pltpu
from jax.experimental.pallas import tpu_sc as plsc

_L = 16


@functools.lru_cache(maxsize=None)
def _build(B, H, W, D):
    C = 2 * D

    mesh = plsc.VectorSubcoreMesh(
        core_axis_name="c", subcore_axis_name="s", num_cores=1)

    @functools.partial(
        pl.kernel,
        mesh=mesh,
        out_type=jax.ShapeDtypeStruct((B, H, W, C), jnp.float32),
        scratch_types=[
            pltpu.VMEM((2, W, C), jnp.float32),
            pltpu.VMEM((2, D), jnp.float32),
            pltpu.SemaphoreType.DMA,
        ],
    )
    def pos_embed(row_hbm, col_hbm, out_hbm, planes, rowv, sem):
        w = lax.axis_index("s")

        pltpu.sync_copy(row_hbm.at[pl.ds(w * 2, 2)], rowv)
        for half in range(2):
            pltpu.sync_copy(
                col_hbm.at[pl.ds(0, W)], planes.at[half, :, pl.ds(0, D)])
            segs = [rowv[half, pl.ds(k * _L, _L)] for k in range(D // _L)]
            for j in range(W):
                for k, v in enumerate(segs):
                    planes[half, j, pl.ds(D + k * _L, _L)] = v

        copies = [
            pltpu.async_copy(planes, out_hbm.at[b, pl.ds(w * 2, 2)], sem)
            for b in range(B)
        ]
        for cp in copies:
            cp.wait()

    return pos_embed


def kernel(x, row_embed, col_embed):
    B = x.shape[0]
    H, W = x.shape[-2], x.shape[-1]
    D = row_embed.shape[1]
    out = _build(B, H, W, D)(row_embed, col_embed)
    return jnp.transpose(out, (0, 3, 1, 2))


# speedup vs baseline: 24.9997x; 1.0841x over previous
import functools

import jax
import jax.numpy as jnp
from jax import lax
from jax.experimental import pallas as pl
from jax.experimental.pallas import tpu as pltpu
from jax.experimental.pallas import tpu_sc as plsc

_L = 16


@functools.lru_cache(maxsize=None)
def _build_sc_lookup(H, W, D):
    C = 2 * D

    mesh = plsc.VectorSubcoreMesh(
        core_axis_name="c", subcore_axis_name="s", num_cores=1)

    @functools.partial(
        pl.kernel,
        mesh=mesh,
        out_type=jax.ShapeDtypeStruct((H, W, C), jnp.float32),
        scratch_types=[
            pltpu.VMEM((2, W, C), jnp.float32),
            pltpu.VMEM((2, D), jnp.float32),
        ],
    )
    def sc_lookup(row_hbm, col_hbm, u_hbm, planes, rowv):
        w = lax.axis_index("s")

        pltpu.sync_copy(row_hbm.at[pl.ds(w * 2, 2)], rowv)
        for half in range(2):
            pltpu.sync_copy(
                col_hbm.at[pl.ds(0, W)], planes.at[half, :, pl.ds(0, D)])
            segs = [rowv[half, pl.ds(k * _L, _L)] for k in range(D // _L)]
            for j in range(W):
                for k, v in enumerate(segs):
                    planes[half, j, pl.ds(D + k * _L, _L)] = v

        pltpu.sync_copy(planes, u_hbm.at[pl.ds(w * 2, 2)])

    return sc_lookup


@functools.lru_cache(maxsize=None)
def _build_tc_broadcast(B, H, W, C):
    def tc_broadcast(u_ref, out_ref):
        out_ref[0] = u_ref[...]

    return pl.pallas_call(
        tc_broadcast,
        grid=(B,),
        in_specs=[
            pl.BlockSpec((H, W, C), lambda b: (0, 0, 0)),
        ],
        out_specs=pl.BlockSpec((1, H, W, C), lambda b: (b, 0, 0, 0)),
        out_shape=jax.ShapeDtypeStruct((B, H, W, C), jnp.float32),
    )


def kernel(x, row_embed, col_embed):
    B = x.shape[0]
    H, W = x.shape[-2], x.shape[-1]
    D = row_embed.shape[1]
    u = _build_sc_lookup(H, W, D)(row_embed, col_embed)
    out = _build_tc_broadcast(B, H, W, 2 * D)(u)
    return jnp.transpose(out, (0, 3, 1, 2))


# speedup vs baseline: 26.1702x; 1.0468x over previous
import functools

import jax
import jax.numpy as jnp
from jax import lax
from jax.experimental import pallas as pl
from jax.experimental.pallas import tpu as pltpu
from jax.experimental.pallas import tpu_sc as plsc

_L = 16


@functools.lru_cache(maxsize=None)
def _build_sc_lookup(H, W, D):
    C = 2 * D

    mesh = plsc.VectorSubcoreMesh(
        core_axis_name="c", subcore_axis_name="s", num_cores=1)

    @functools.partial(
        pl.kernel,
        mesh=mesh,
        out_type=jax.ShapeDtypeStruct((H, W, C), jnp.float32),
        scratch_types=[
            pltpu.VMEM((2, W, C), jnp.float32),
            pltpu.VMEM((2, D), jnp.float32),
        ],
    )
    def sc_lookup(row_hbm, col_hbm, u_hbm, planes, rowv):
        w = lax.axis_index("s")

        pltpu.sync_copy(row_hbm.at[pl.ds(w * 2, 2)], rowv)
        for half in range(2):
            pltpu.sync_copy(
                col_hbm.at[pl.ds(0, W)], planes.at[half, :, pl.ds(0, D)])
            segs = [rowv[half, pl.ds(k * _L, _L)] for k in range(D // _L)]
            for j in range(W):
                for k, v in enumerate(segs):
                    planes[half, j, pl.ds(D + k * _L, _L)] = v

        pltpu.sync_copy(planes, u_hbm.at[pl.ds(w * 2, 2)])

    return sc_lookup


@functools.lru_cache(maxsize=None)
def _build_tc_broadcast(B, H, W, C):
    def tc_broadcast(u_ref, out_ref, sem):
        copies = [
            pltpu.make_async_copy(u_ref, out_ref.at[b], sem)
            for b in range(B)
        ]
        for cp in copies:
            cp.start()
        for cp in copies:
            cp.wait()

    return pl.pallas_call(
        tc_broadcast,
        in_specs=[pl.BlockSpec(memory_space=pltpu.VMEM)],
        out_specs=pl.BlockSpec(memory_space=pl.ANY),
        out_shape=jax.ShapeDtypeStruct((B, H, W, C), jnp.float32),
        scratch_shapes=[pltpu.SemaphoreType.DMA],
    )


def kernel(x, row_embed, col_embed):
    B = x.shape[0]
    H, W = x.shape[-2], x.shape[-1]
    D = row_embed.shape[1]
    u = _build_sc_lookup(H, W, D)(row_embed, col_embed)
    out = _build_tc_broadcast(B, H, W, 2 * D)(u)
    return jnp.transpose(out, (0, 3, 1, 2))


# speedup vs baseline: 29.0373x; 1.1096x over previous
import functools

import jax
import jax.numpy as jnp
from jax import lax
from jax.experimental import pallas as pl
from jax.experimental.pallas import tpu as pltpu
from jax.experimental.pallas import tpu_sc as plsc

_L = 16


@functools.lru_cache(maxsize=None)
def _build(B, H, W, D):
    C = 2 * D

    mesh = plsc.VectorSubcoreMesh(core_axis_name="c", subcore_axis_name="s")

    @functools.partial(
        pl.kernel,
        mesh=mesh,
        out_type=jax.ShapeDtypeStruct((B, H, W, C), jnp.float32),
        scratch_types=[
            pltpu.VMEM((W, C), jnp.float32),
            pltpu.VMEM((1, D), jnp.float32),
            pltpu.SemaphoreType.DMA,
        ],
    )
    def pos_embed(row_hbm, col_hbm, out_hbm, plane, rowv, sem):
        i = lax.axis_index("c") * 16 + lax.axis_index("s")

        pltpu.sync_copy(col_hbm.at[pl.ds(0, W)], plane.at[:, pl.ds(0, D)])
        pltpu.sync_copy(row_hbm.at[pl.ds(i, 1)], rowv)

        segs = [rowv[0, pl.ds(k * _L, _L)] for k in range(D // _L)]
        for j in range(W):
            for k, v in enumerate(segs):
                plane[j, pl.ds(D + k * _L, _L)] = v

        copies = [
            pltpu.async_copy(plane, out_hbm.at[b, i], sem) for b in range(B)
        ]
        for cp in copies:
            cp.wait()

    return pos_embed


def kernel(x, row_embed, col_embed):
    B = x.shape[0]
    H, W = x.shape[-2], x.shape[-1]
    D = row_embed.shape[1]
    out = _build(B, H, W, D)(row_embed, col_embed)
    return jnp.transpose(out, (0, 3, 1, 2))
